# Initial kernel scaffold; baseline (speedup 1.0000x reference)
import jax
import jax.numpy as jnp
from jax.experimental import pallas as pl


def kernel(xyz, new_xyz, features):
    raise NotImplementedError("write your pallas kernel here")


# speedup vs baseline: 6.9877x; 6.9877x over previous
import functools

import jax
import jax.numpy as jnp
from jax import lax
from jax.experimental import pallas as pl
from jax.experimental.pallas import tpu as pltpu
from jax.experimental.pallas import tpu_sc as plsc

RADIUS = 0.1
NSAMPLE = 32

B = 4
N = 8192
NPOINT = 2048
C = 128

NC = 2
NS = 16
NW = NC * NS
L = 16

QPW = NPOINT // NW


def _ball_query_body(xyz_hbm, newxyz_hbm, idx_hbm, pts, qc, idxbuf, dsem):
    wid = lax.axis_index("s") * NC + lax.axis_index("c")
    qbase = wid * QPW
    r2 = jnp.float32(RADIUS * RADIUS)
    lanes = lax.iota(jnp.int32, L)
    zeros16 = jnp.zeros((L,), jnp.int32)

    for b in range(B):
        pltpu.sync_copy(xyz_hbm.at[b], pts)
        for d in range(3):
            pltpu.sync_copy(newxyz_hbm.at[b, d, pl.ds(qbase, QPW)],
                            qc.at[pl.ds(d * QPW, QPW)])

        def per_query(qi, _, b=b):
            qsplat = jnp.full((L,), qi, jnp.int32)
            qx = plsc.load_gather(qc, [qsplat])
            qy = plsc.load_gather(qc, [qsplat + QPW])
            qz = plsc.load_gather(qc, [qsplat + 2 * QPW])
            obase = qi * NSAMPLE

            def scan_body(j, cvec):
                base = j * L
                vx = pts[0, pl.ds(base, L)]
                vy = pts[1, pl.ds(base, L)]
                vz = pts[2, pl.ds(base, L)]
                dx = vx - qx
                dy = vy - qy
                dz = vz - qz
                d = dx * dx + dy * dy + dz * dz
                m = d < r2
                mi = m.astype(jnp.int32)
                excl = jnp.cumsum(mi) - mi
                dest = cvec + excl
                ok = m & (dest < NSAMPLE)
                plsc.store_scatter(idxbuf, [obase + dest], base + lanes,
                                   mask=ok)
                cnt = plsc.all_reduce_population_count(m)
                return cvec + cnt

            cvec = lax.fori_loop(0, N // L, scan_body, zeros16)

            obasev = jnp.full((L,), obase, jnp.int32)
            first = plsc.load_gather(idxbuf, [obasev])
            first = jnp.where(cvec > 0, first, 0)
            v0 = plsc.load_gather(idxbuf, [obasev + lanes])
            v1 = plsc.load_gather(idxbuf, [obasev + lanes + L])
            v0 = jnp.where(lanes < cvec, v0, first)
            v1 = jnp.where(lanes + L < cvec, v1, first)
            plsc.store_scatter(idxbuf, [obasev + lanes], v0)
            plsc.store_scatter(idxbuf, [obasev + lanes + L], v1)
            return 0

        lax.fori_loop(0, QPW, per_query, 0)
        pltpu.sync_copy(idxbuf,
                        idx_hbm.at[b, pl.ds(qbase * NSAMPLE, QPW * NSAMPLE)])


def _group_body(xyzt_hbm, newxyzt_hbm, feat_hbm, idx_hbm, out_hbm,
                idxv, frow, qrow, stage, dsem):
    wid = lax.axis_index("s") * NC + lax.axis_index("c")
    CW = 512 * NSAMPLE
    CHQ = NPOINT // 512

    def gather_row(src_is_xyz, ch, b):
        for chunk in range(CHQ):
            def gi(i, _):
                iv = idxv[pl.ds(chunk * CW + i * L, L)]
                g = plsc.load_gather(frow, [iv])
                if src_is_xyz:
                    qs = plsc.load_gather(
                        qrow,
                        [jnp.full((L,), chunk * 512 + i // 2, jnp.int32)])
                    g = g - qs
                stage[pl.ds(i * L, L)] = g
                return 0

            lax.fori_loop(0, CW // L, gi, 0)
            out_ch = ch if src_is_xyz else ch + 3
            pltpu.sync_copy(stage, out_hbm.at[b, out_ch,
                                              pl.ds(chunk * CW, CW)])

    for b in range(B):
        pltpu.sync_copy(idx_hbm.at[b], idxv)

        for k in range(C // NW):
            ch = wid * (C // NW) + k
            pltpu.sync_copy(feat_hbm.at[b, ch], frow)
            gather_row(False, ch, b)

        @pl.when(wid < 3)
        def _():
            d = wid
            pltpu.sync_copy(xyzt_hbm.at[b, d], frow)
            pltpu.sync_copy(newxyzt_hbm.at[b, d], qrow)
            gather_row(True, d, b)


@jax.jit
def _run(xyz_t, new_t, features):
    mesh = plsc.VectorSubcoreMesh(core_axis_name="c", subcore_axis_name="s")
    cparams = pltpu.CompilerParams(needs_layout_passes=False)

    ball = pl.kernel(
        _ball_query_body,
        compiler_params=cparams,
        out_type=jax.ShapeDtypeStruct((B, NPOINT * NSAMPLE), jnp.int32),
        mesh=mesh,
        scratch_types=[
            pltpu.VMEM((3, N), jnp.float32),
            pltpu.VMEM((3 * QPW,), jnp.float32),
            pltpu.VMEM((QPW * NSAMPLE,), jnp.int32),
            pltpu.SemaphoreType.DMA,
        ],
    )
    idx = ball(xyz_t, new_t)

    group = pl.kernel(
        _group_body,
        compiler_params=cparams,
        out_type=jax.ShapeDtypeStruct((B, 3 + C, NPOINT * NSAMPLE),
                                      jnp.float32),
        mesh=mesh,
        scratch_types=[
            pltpu.VMEM((NPOINT * NSAMPLE,), jnp.int32),
            pltpu.VMEM((N,), jnp.float32),
            pltpu.VMEM((NPOINT,), jnp.float32),
            pltpu.VMEM((512 * NSAMPLE,), jnp.float32),
            pltpu.SemaphoreType.DMA,
        ],
    )
    out = group(xyz_t, new_t, features, idx)
    return out.reshape(B, 3 + C, NPOINT, NSAMPLE)


def kernel(xyz, new_xyz, features):
    xyz_t = jnp.transpose(xyz, (0, 2, 1))
    new_t = jnp.transpose(new_xyz, (0, 2, 1))
    return _run(xyz_t, new_t, features)
